# baseline (speedup 1.0000x reference)
import functools

import jax
import jax.numpy as jnp
from jax import lax
from jax.experimental import pallas as pl
from jax.experimental.pallas import tpu as pltpu
from jax.experimental.pallas import tpu_sc as plsc

E, K, N = 16384, 8, 50000
D_IMG, D_TXT, H = 256, 512, 512
THRESHOLD = 0.5

NC, NS, L = 2, 16, 16
NW = NC * NS
CHUNK = 8
CK = CHUNK * K
NBUF = 2
NSPLIT = 4
EBLK = E // NSPLIT
EPW = EBLK // NW
NCHUNK = EPW // CHUNK


def _pool_body(he_hbm, img_hbm, txt_hbm, out_img_hbm, out_txt_hbm,
               idx_all, *scr):
    rows_img = scr[0:NBUF]
    rows_txt = scr[NBUF:2 * NBUF]
    pooled_img = scr[2 * NBUF:3 * NBUF]
    pooled_txt = scr[3 * NBUF:4 * NBUF]
    sem_i = scr[4 * NBUF:5 * NBUF]
    sem_t = scr[5 * NBUF:6 * NBUF]
    sem_oi = scr[6 * NBUF:7 * NBUF]
    sem_ot = scr[7 * NBUF:8 * NBUF]

    wid = lax.axis_index("s") * NC + lax.axis_index("c")
    base_e = wid * EPW
    pltpu.sync_copy(he_hbm.at[pl.ds(base_e * K, EPW * K)], idx_all)

    def start(c, b):
        idx = idx_all.at[pl.ds(c * CK, CK)]
        pltpu.async_copy(img_hbm.at[idx], rows_img[b], sem_i[b])
        pltpu.async_copy(txt_hbm.at[idx], rows_txt[b], sem_t[b])

    def finish(c, b):
        e0 = base_e + c * CHUNK
        idx = idx_all.at[pl.ds(c * CK, CK)]
        pltpu.make_async_copy(img_hbm.at[idx], rows_img[b], sem_i[b]).wait()
        pltpu.make_async_copy(txt_hbm.at[idx], rows_txt[b], sem_t[b]).wait()

        @pl.when(c >= NBUF)
        def _():
            pltpu.make_async_copy(
                pooled_img[b], out_img_hbm.at[pl.ds(e0, CHUNK), :],
                sem_oi[b]).wait()
            pltpu.make_async_copy(
                pooled_txt[b], out_txt_hbm.at[pl.ds(e0, CHUNK), :],
                sem_ot[b]).wait()

        def edge_body(e, carry):
            r0 = e * K
            for v in range(D_IMG // L):
                sl = pl.ds(v * L, L)
                acc = rows_img[b][r0, sl]
                for k in range(1, K):
                    acc = acc + rows_img[b][r0 + k, sl]
                pooled_img[b][e, sl] = acc * (1.0 / K)
            for v in range(D_TXT // L):
                sl = pl.ds(v * L, L)
                acc = rows_txt[b][r0, sl]
                for k in range(1, K):
                    acc = acc + rows_txt[b][r0 + k, sl]
                pooled_txt[b][e, sl] = acc * (1.0 / K)
            return carry

        lax.fori_loop(0, CHUNK, edge_body, 0)
        pltpu.async_copy(pooled_img[b], out_img_hbm.at[pl.ds(e0, CHUNK), :],
                         sem_oi[b])
        pltpu.async_copy(pooled_txt[b], out_txt_hbm.at[pl.ds(e0, CHUNK), :],
                         sem_ot[b])

    for b in range(NBUF):
        start(b, b)

    def body(i, carry):
        c0 = NBUF * i
        for b in range(NBUF):
            finish(c0 + b, b)

            @pl.when(i < NCHUNK // NBUF - 1)
            def _():
                start(c0 + b + NBUF, b)

        return carry

    lax.fori_loop(0, NCHUNK // NBUF, body, 0)

    for b in range(NBUF):
        c = NCHUNK - NBUF + b
        e0 = base_e + c * CHUNK
        pltpu.make_async_copy(
            pooled_img[b], out_img_hbm.at[pl.ds(e0, CHUNK), :],
            sem_oi[b]).wait()
        pltpu.make_async_copy(
            pooled_txt[b], out_txt_hbm.at[pl.ds(e0, CHUNK), :],
            sem_ot[b]).wait()


@functools.cache
def _get_pool():
    return pl.kernel(
        _pool_body,
        out_type=[
            jax.ShapeDtypeStruct((EBLK, D_IMG), jnp.float32),
            jax.ShapeDtypeStruct((EBLK, D_TXT), jnp.float32),
        ],
        mesh=plsc.VectorSubcoreMesh(
            core_axis_name="c", subcore_axis_name="s",
            num_cores=NC, num_subcores=NS),
        scratch_types=(
            [pltpu.VMEM((EPW * K,), jnp.int32)]
            + [pltpu.VMEM((CK, D_IMG), jnp.float32)] * NBUF
            + [pltpu.VMEM((CK, D_TXT), jnp.float32)] * NBUF
            + [pltpu.VMEM((CHUNK, D_IMG), jnp.float32)] * NBUF
            + [pltpu.VMEM((CHUNK, D_TXT), jnp.float32)] * NBUF
            + [pltpu.SemaphoreType.DMA] * (4 * NBUF)
        ),
    )


BE = 2048


def _mlp_body(pi_ref, pt_ref, w1i_ref, b1i_ref, w2i_ref,
              w1t_ref, b1t_ref, w2t_ref, scal_ref,
              scores_ref, mask_ref):
    hi = jnp.maximum(
        jnp.dot(pi_ref[...], w1i_ref[...],
                preferred_element_type=jnp.float32) + b1i_ref[...], 0.0)
    si = jnp.dot(hi, w2i_ref[...],
                 preferred_element_type=jnp.float32)[:, 0] + scal_ref[0]
    ht = jnp.maximum(
        jnp.dot(pt_ref[...], w1t_ref[...],
                preferred_element_type=jnp.float32) + b1t_ref[...], 0.0)
    st = jnp.dot(ht, w2t_ref[...],
                 preferred_element_type=jnp.float32)[:, 0] + scal_ref[1]
    e_score = scal_ref[2] * si + scal_ref[3] * st
    scores = jax.nn.sigmoid(e_score)
    scores_ref[...] = scores
    mask_ref[...] = scores > THRESHOLD


_mlp = pl.pallas_call(
    _mlp_body,
    grid=(EBLK // BE,),
    in_specs=[
        pl.BlockSpec((BE, D_IMG), lambda i: (i, 0)),
        pl.BlockSpec((BE, D_TXT), lambda i: (i, 0)),
        pl.BlockSpec((D_IMG, H), lambda i: (0, 0)),
        pl.BlockSpec((1, H), lambda i: (0, 0)),
        pl.BlockSpec((H, 1), lambda i: (0, 0)),
        pl.BlockSpec((D_TXT, H), lambda i: (0, 0)),
        pl.BlockSpec((1, H), lambda i: (0, 0)),
        pl.BlockSpec((H, 1), lambda i: (0, 0)),
        pl.BlockSpec(memory_space=pltpu.SMEM),
    ],
    out_specs=[
        pl.BlockSpec((BE,), lambda i: (i,)),
        pl.BlockSpec((BE,), lambda i: (i,)),
    ],
    out_shape=[
        jax.ShapeDtypeStruct((EBLK,), jnp.float32),
        jax.ShapeDtypeStruct((EBLK,), jnp.bool_),
    ],
)


def kernel(hyperedges, features_image, features_text,
           W1_image, b1_image, W2_image, b2_image,
           W1_text, b1_text, W2_text, b2_text, alpha):
    he = jnp.asarray(hyperedges, jnp.int32).reshape(E * K)

    w = jax.nn.softmax(alpha, axis=0)
    scal = jnp.stack([b2_image[0], b2_text[0], w[0], w[1]])
    b1i = b1_image.reshape(1, H)
    b1t = b1_text.reshape(1, H)

    pool = _get_pool()
    scores_blocks, mask_blocks = [], []
    for j in range(NSPLIT):
        he_j = lax.dynamic_slice_in_dim(he, j * EBLK * K, EBLK * K)
        pooled_img, pooled_txt = pool(he_j, features_image, features_text)
        s_j, m_j = _mlp(pooled_img, pooled_txt,
                        W1_image, b1i, W2_image,
                        W1_text, b1t, W2_text, scal)
        scores_blocks.append(s_j)
        mask_blocks.append(m_j)

    scores = jnp.concatenate(scores_blocks)
    mask = jnp.concatenate(mask_blocks)
    return (mask, scores)


# speedup vs baseline: 1.0557x; 1.0557x over previous
import functools

import jax
import jax.numpy as jnp
from jax import lax
from jax.experimental import pallas as pl
from jax.experimental.pallas import tpu as pltpu
from jax.experimental.pallas import tpu_sc as plsc

E, K, N = 16384, 8, 50000
D_IMG, D_TXT, H = 256, 512, 512
THRESHOLD = 0.5

NC, NS, L = 2, 16, 16
NW = NC * NS
CHUNK = 8
CK = CHUNK * K
NBUF = 2
NSPLIT = 1
EBLK = E // NSPLIT
EPW = EBLK // NW
NCHUNK = EPW // CHUNK


def _pool_body(he_hbm, img_hbm, txt_hbm, out_img_hbm, out_txt_hbm,
               idx_all, *scr):
    rows_img = scr[0:NBUF]
    rows_txt = scr[NBUF:2 * NBUF]
    pooled_img = scr[2 * NBUF:3 * NBUF]
    pooled_txt = scr[3 * NBUF:4 * NBUF]
    sem_i = scr[4 * NBUF:5 * NBUF]
    sem_t = scr[5 * NBUF:6 * NBUF]
    sem_oi = scr[6 * NBUF:7 * NBUF]
    sem_ot = scr[7 * NBUF:8 * NBUF]

    wid = lax.axis_index("s") * NC + lax.axis_index("c")
    base_e = wid * EPW
    pltpu.sync_copy(he_hbm.at[pl.ds(base_e * K, EPW * K)], idx_all)

    def start(c, b):
        idx = idx_all.at[pl.ds(c * CK, CK)]
        pltpu.async_copy(img_hbm.at[idx], rows_img[b], sem_i[b])
        pltpu.async_copy(txt_hbm.at[idx], rows_txt[b], sem_t[b])

    def finish(c, b):
        e0 = base_e + c * CHUNK
        idx = idx_all.at[pl.ds(c * CK, CK)]
        pltpu.make_async_copy(img_hbm.at[idx], rows_img[b], sem_i[b]).wait()
        pltpu.make_async_copy(txt_hbm.at[idx], rows_txt[b], sem_t[b]).wait()

        @pl.when(c >= NBUF)
        def _():
            pltpu.make_async_copy(
                pooled_img[b], out_img_hbm.at[pl.ds(e0, CHUNK), :],
                sem_oi[b]).wait()
            pltpu.make_async_copy(
                pooled_txt[b], out_txt_hbm.at[pl.ds(e0, CHUNK), :],
                sem_ot[b]).wait()

        def edge_body(e, carry):
            r0 = e * K
            for v in range(D_IMG // L):
                sl = pl.ds(v * L, L)
                acc = rows_img[b][r0, sl]
                for k in range(1, K):
                    acc = acc + rows_img[b][r0 + k, sl]
                pooled_img[b][e, sl] = acc * (1.0 / K)
            for v in range(D_TXT // L):
                sl = pl.ds(v * L, L)
                acc = rows_txt[b][r0, sl]
                for k in range(1, K):
                    acc = acc + rows_txt[b][r0 + k, sl]
                pooled_txt[b][e, sl] = acc * (1.0 / K)
            return carry

        lax.fori_loop(0, CHUNK, edge_body, 0)
        pltpu.async_copy(pooled_img[b], out_img_hbm.at[pl.ds(e0, CHUNK), :],
                         sem_oi[b])
        pltpu.async_copy(pooled_txt[b], out_txt_hbm.at[pl.ds(e0, CHUNK), :],
                         sem_ot[b])

    for b in range(NBUF):
        start(b, b)

    def body(i, carry):
        c0 = NBUF * i
        for b in range(NBUF):
            finish(c0 + b, b)

            @pl.when(i < NCHUNK // NBUF - 1)
            def _():
                start(c0 + b + NBUF, b)

        return carry

    lax.fori_loop(0, NCHUNK // NBUF, body, 0)

    for b in range(NBUF):
        c = NCHUNK - NBUF + b
        e0 = base_e + c * CHUNK
        pltpu.make_async_copy(
            pooled_img[b], out_img_hbm.at[pl.ds(e0, CHUNK), :],
            sem_oi[b]).wait()
        pltpu.make_async_copy(
            pooled_txt[b], out_txt_hbm.at[pl.ds(e0, CHUNK), :],
            sem_ot[b]).wait()


@functools.cache
def _get_pool():
    return pl.kernel(
        _pool_body,
        out_type=[
            jax.ShapeDtypeStruct((EBLK, D_IMG), jnp.float32),
            jax.ShapeDtypeStruct((EBLK, D_TXT), jnp.float32),
        ],
        mesh=plsc.VectorSubcoreMesh(
            core_axis_name="c", subcore_axis_name="s",
            num_cores=NC, num_subcores=NS),
        scratch_types=(
            [pltpu.VMEM((EPW * K,), jnp.int32)]
            + [pltpu.VMEM((CK, D_IMG), jnp.float32)] * NBUF
            + [pltpu.VMEM((CK, D_TXT), jnp.float32)] * NBUF
            + [pltpu.VMEM((CHUNK, D_IMG), jnp.float32)] * NBUF
            + [pltpu.VMEM((CHUNK, D_TXT), jnp.float32)] * NBUF
            + [pltpu.SemaphoreType.DMA] * (4 * NBUF)
        ),
    )


BE = 4096


def _mlp_body(pi_ref, pt_ref, w1i_ref, b1i_ref, w2i_ref,
              w1t_ref, b1t_ref, w2t_ref, scal_ref,
              scores_ref, mask_ref):
    hi = jnp.maximum(
        jnp.dot(pi_ref[...], w1i_ref[...],
                preferred_element_type=jnp.float32) + b1i_ref[...], 0.0)
    si = jnp.dot(hi, w2i_ref[...],
                 preferred_element_type=jnp.float32)[:, 0] + scal_ref[0]
    ht = jnp.maximum(
        jnp.dot(pt_ref[...], w1t_ref[...],
                preferred_element_type=jnp.float32) + b1t_ref[...], 0.0)
    st = jnp.dot(ht, w2t_ref[...],
                 preferred_element_type=jnp.float32)[:, 0] + scal_ref[1]
    e_score = scal_ref[2] * si + scal_ref[3] * st
    scores = jax.nn.sigmoid(e_score)
    scores_ref[...] = scores
    mask_ref[...] = scores > THRESHOLD


_mlp = pl.pallas_call(
    _mlp_body,
    grid=(EBLK // BE,),
    in_specs=[
        pl.BlockSpec((BE, D_IMG), lambda i: (i, 0)),
        pl.BlockSpec((BE, D_TXT), lambda i: (i, 0)),
        pl.BlockSpec((D_IMG, H), lambda i: (0, 0)),
        pl.BlockSpec((1, H), lambda i: (0, 0)),
        pl.BlockSpec((H, 1), lambda i: (0, 0)),
        pl.BlockSpec((D_TXT, H), lambda i: (0, 0)),
        pl.BlockSpec((1, H), lambda i: (0, 0)),
        pl.BlockSpec((H, 1), lambda i: (0, 0)),
        pl.BlockSpec(memory_space=pltpu.SMEM),
    ],
    out_specs=[
        pl.BlockSpec((BE,), lambda i: (i,)),
        pl.BlockSpec((BE,), lambda i: (i,)),
    ],
    out_shape=[
        jax.ShapeDtypeStruct((EBLK,), jnp.float32),
        jax.ShapeDtypeStruct((EBLK,), jnp.bool_),
    ],
)


def kernel(hyperedges, features_image, features_text,
           W1_image, b1_image, W2_image, b2_image,
           W1_text, b1_text, W2_text, b2_text, alpha):
    he = jnp.asarray(hyperedges, jnp.int32).reshape(E * K)

    w = jax.nn.softmax(alpha, axis=0)
    scal = jnp.stack([b2_image[0], b2_text[0], w[0], w[1]])
    b1i = b1_image.reshape(1, H)
    b1t = b1_text.reshape(1, H)

    pool = _get_pool()
    scores_blocks, mask_blocks = [], []
    for j in range(NSPLIT):
        he_j = lax.dynamic_slice_in_dim(he, j * EBLK * K, EBLK * K)
        pooled_img, pooled_txt = pool(he_j, features_image, features_text)
        s_j, m_j = _mlp(pooled_img, pooled_txt,
                        W1_image, b1i, W2_image,
                        W1_text, b1t, W2_text, scal)
        scores_blocks.append(s_j)
        mask_blocks.append(m_j)

    scores = jnp.concatenate(scores_blocks)
    mask = jnp.concatenate(mask_blocks)
    return (mask, scores)
